# baseline (device time: 16904 ns/iter reference)
import jax
import jax.numpy as jnp
from jax import lax
from jax.experimental import pallas as pl
from jax.experimental.pallas import tpu as pltpu

T = 256
D = 512
V_SHARD = 4096
NC = 8
VC = V_SHARD // NC


def kernel(x, W, labels):
    labels2d = labels.reshape(T, 1)

    def body(
        x_ref,
        w_hbm,
        lab_ref,
        out_ref,
        wbuf,
        send_buf,
        recv_buf,
        copy_sems,
        send_sem,
        recv_sem,
    ):
        my_x = lax.axis_index("x")
        my_y = lax.axis_index("y")
        my_z = lax.axis_index("z")
        nbr = (my_x, my_y, 1 - my_z)

        barrier_sem = pltpu.get_barrier_semaphore()
        pl.semaphore_signal(
            barrier_sem, inc=1, device_id=nbr, device_id_type=pl.DeviceIdType.MESH
        )

        def copy_chunk(j):
            return pltpu.make_async_copy(
                w_hbm.at[:, pl.ds(j * VC, VC)], wbuf.at[j % 2], copy_sems.at[j % 2]
            )

        copy_chunk(0).start()
        xb = x_ref[:, :].astype(jnp.bfloat16)
        lab = lab_ref[:, :] - my_z * V_SHARD
        cols = lax.broadcasted_iota(jnp.int32, (T, VC), 1)

        s_acc = jnp.zeros((T, 1), jnp.float32)
        t_acc = jnp.zeros((T, 1), jnp.float32)
        for j in range(NC):
            if j + 1 < NC:
                copy_chunk(j + 1).start()
            copy_chunk(j).wait()
            logits = jnp.dot(
                xb, wbuf[j % 2].astype(jnp.bfloat16), preferred_element_type=jnp.float32
            )
            s_acc += jnp.sum(jnp.exp(logits), axis=1, keepdims=True)
            t_acc += jnp.sum(
                jnp.where(cols == lab - j * VC, logits, 0.0), axis=1, keepdims=True
            )

        send_buf[:, :] = jnp.concatenate([s_acc, t_acc], axis=1)
        pl.semaphore_wait(barrier_sem, 1)
        rdma = pltpu.make_async_remote_copy(
            src_ref=send_buf,
            dst_ref=recv_buf,
            send_sem=send_sem,
            recv_sem=recv_sem,
            device_id=nbr,
            device_id_type=pl.DeviceIdType.MESH,
        )
        rdma.start()
        rdma.wait()
        sg = s_acc + recv_buf[:, 0:1]
        tg = t_acc + recv_buf[:, 1:2]
        out_ref[:] = (jnp.log(sg) - tg)[:, 0]

    out = pl.pallas_call(
        body,
        out_shape=jax.ShapeDtypeStruct((T,), jnp.float32),
        in_specs=[
            pl.BlockSpec(memory_space=pltpu.VMEM),
            pl.BlockSpec(memory_space=pltpu.MemorySpace.HBM),
            pl.BlockSpec(memory_space=pltpu.VMEM),
        ],
        out_specs=pl.BlockSpec(memory_space=pltpu.VMEM),
        scratch_shapes=[
            pltpu.VMEM((2, D, VC), jnp.float32),
            pltpu.VMEM((T, 2), jnp.float32),
            pltpu.VMEM((T, 2), jnp.float32),
            pltpu.SemaphoreType.DMA((2,)),
            pltpu.SemaphoreType.DMA,
            pltpu.SemaphoreType.DMA,
        ],
        compiler_params=pltpu.CompilerParams(collective_id=0),
    )(x, W, labels2d)
    return out


# device time: 10462 ns/iter; 1.6158x vs baseline; 1.6158x over previous
import jax
import jax.numpy as jnp
from jax import lax
from jax.experimental import pallas as pl
from jax.experimental.pallas import tpu as pltpu

T = 256
D = 512
V_SHARD = 4096
NC = 8
VC = V_SHARD // NC


def kernel(x, W, labels):
    x = pltpu.with_memory_space_constraint(x, pltpu.MemorySpace.HBM)
    W = pltpu.with_memory_space_constraint(W, pltpu.MemorySpace.HBM)
    labels = pltpu.with_memory_space_constraint(labels, pltpu.MemorySpace.HBM)

    def body(
        x_hbm,
        w_hbm,
        lab_hbm,
        out_ref,
        xv,
        labv,
        wbuf,
        send_buf,
        recv_buf,
        xsem,
        labsem,
        wsems,
        send_sem,
        recv_sem,
    ):
        my_x = lax.axis_index("x")
        my_y = lax.axis_index("y")
        my_z = lax.axis_index("z")
        nbr = (my_x, my_y, 1 - my_z)

        barrier_sem = pltpu.get_barrier_semaphore()
        pl.semaphore_signal(
            barrier_sem, inc=1, device_id=nbr, device_id_type=pl.DeviceIdType.MESH
        )

        cw0 = pltpu.make_async_copy(
            w_hbm.at[pl.ds(0, D // 2), :], wbuf.at[pl.ds(0, D // 2), :], wsems.at[0]
        )
        cw1 = pltpu.make_async_copy(
            w_hbm.at[pl.ds(D // 2, D // 2), :],
            wbuf.at[pl.ds(D // 2, D // 2), :],
            wsems.at[1],
        )
        cx = pltpu.make_async_copy(x_hbm, xv, xsem)
        cl = pltpu.make_async_copy(lab_hbm, labv, labsem)
        cw0.start()
        cw1.start()
        cx.start()
        cl.start()
        cx.wait()
        cl.wait()

        xb = xv[:, :]
        lab = labv[:].reshape(T, 1) - my_z * V_SHARD
        cols = lax.broadcasted_iota(jnp.int32, (T, VC), 1)

        s_acc = jnp.zeros((T, 1), jnp.float32)
        t_acc = jnp.zeros((T, 1), jnp.float32)
        cw0.wait()
        cw1.wait()
        for j in range(NC):
            wj = wbuf[:, pl.ds(j * VC, VC)]
            logits = jnp.dot(xb, wj, preferred_element_type=jnp.float32)
            s_acc += jnp.sum(jnp.exp(logits), axis=1, keepdims=True)
            t_acc += jnp.sum(
                jnp.where(cols == lab - j * VC, logits, 0.0), axis=1, keepdims=True
            )

        s_row = s_acc[:, 0].reshape(1, T)
        t_row = t_acc[:, 0].reshape(1, T)
        send_buf[0:1, :] = s_row
        send_buf[1:2, :] = t_row
        pl.semaphore_wait(barrier_sem, 1)
        rdma = pltpu.make_async_remote_copy(
            src_ref=send_buf,
            dst_ref=recv_buf,
            send_sem=send_sem,
            recv_sem=recv_sem,
            device_id=nbr,
            device_id_type=pl.DeviceIdType.MESH,
        )
        rdma.start()
        rdma.wait()
        sg = s_row + recv_buf[0:1, :]
        tg = t_row + recv_buf[1:2, :]
        out_ref[:] = (jnp.log(sg) - tg).reshape(T)

    out = pl.pallas_call(
        body,
        out_shape=jax.ShapeDtypeStruct((T,), jnp.float32),
        in_specs=[
            pl.BlockSpec(memory_space=pltpu.MemorySpace.HBM),
            pl.BlockSpec(memory_space=pltpu.MemorySpace.HBM),
            pl.BlockSpec(memory_space=pltpu.MemorySpace.HBM),
        ],
        out_specs=pl.BlockSpec(memory_space=pltpu.VMEM),
        scratch_shapes=[
            pltpu.VMEM((T, D), jnp.float32),
            pltpu.VMEM((T,), jnp.int32),
            pltpu.VMEM((D, V_SHARD), jnp.float32),
            pltpu.VMEM((2, T), jnp.float32),
            pltpu.VMEM((2, T), jnp.float32),
            pltpu.SemaphoreType.DMA,
            pltpu.SemaphoreType.DMA,
            pltpu.SemaphoreType.DMA((2,)),
            pltpu.SemaphoreType.DMA,
            pltpu.SemaphoreType.DMA,
        ],
        compiler_params=pltpu.CompilerParams(collective_id=0),
    )(x, W, labels)
    return out
